# baseline (speedup 1.0000x reference)
import jax
import jax.numpy as jnp
from jax.experimental import pallas as pl
from jax.experimental.pallas import tpu as pltpu

SEQ = 2048
EMBED = 1024
HEADS = 16
HEAD_DIM = EMBED // HEADS
TILE = 256
NUM_TILES = SEQ // TILE

_DN = (((1,), (1,)), ((), ()))


def _dotT(a, b):
    return jax.lax.dot_general(a, b, _DN, preferred_element_type=jnp.float32)


def _body(x_hbm, seg_ref, pos_ref, wq_hbm, bq_ref, wk_hbm, bk_ref,
          wv_hbm, bv_ref, wo_hbm, bo_ref, out_ref,
          x_scr, wq_scr, wkv_scr, wo_scr,
          k_scr, v2_scr, num_scr, acc_ref, vseg_ref, sems):
    pos = pos_ref[0]
    seg = seg_ref[...]
    flat_idx = (jax.lax.broadcasted_iota(jnp.int32, seg.shape, 0) * 128
                + jax.lax.broadcasted_iota(jnp.int32, seg.shape, 1))
    sid = jnp.sum(jnp.where(flat_idx == pos, seg, 0))
    start = jnp.sum((seg < sid).astype(jnp.int32))
    end = jnp.sum((seg <= sid).astype(jnp.int32))
    length = end - start
    nt = jnp.minimum((length + 7 + TILE - 1) // TILE, NUM_TILES)
    src0 = 8 * jnp.minimum(start // 8, SEQ // 8 - nt * (TILE // 8))
    ta0 = start // TILE
    ta1 = (end - 1) // TILE + 1
    nseg = ta1 - ta0

    def xcp(i):
        t = (ta0 + i) % NUM_TILES
        return pltpu.make_async_copy(
            x_hbm.at[pl.ds(t * TILE, TILE), :],
            x_scr.at[pl.ds(t * TILE, TILE), :], sems.at[i])

    cp_wk = pltpu.make_async_copy(wk_hbm, wkv_scr.at[pl.ds(0, EMBED), :],
                                  sems.at[8])
    cp_wv = pltpu.make_async_copy(wv_hbm, wkv_scr.at[pl.ds(EMBED, EMBED), :],
                                  sems.at[9])
    cp_wq = pltpu.make_async_copy(wq_hbm, wq_scr, sems.at[10])
    cp_wo = pltpu.make_async_copy(wo_hbm, wo_scr, sems.at[11])

    xcp(0).start()
    xcp(1).start()
    cp_wk.start()
    cp_wv.start()
    cp_wq.start()
    for i in range(2, NUM_TILES):
        xcp(i).start()
    cp_wo.start()

    acc_ref[...] = jnp.zeros_like(acc_ref)
    vseg_ref[...] = jnp.zeros_like(vseg_ref)

    jax.lax.fori_loop(0, nseg, lambda i, c: (xcp(i).wait(), c)[1], 0)
    cp_wk.wait()
    cp_wv.wait()

    def kv_body(t, _):
        rows = x_scr[pl.ds(src0 + t * TILE, TILE), :]
        kvt = _dotT(rows, wkv_scr[...])
        kt = kvt[:, 0:EMBED] + bk_ref[...]
        vt = kvt[:, EMBED:2 * EMBED] + bv_ref[...]
        k_scr[pl.ds(t * TILE, TILE), :] = kt
        gidx = (src0 + t * TILE
                + jax.lax.broadcasted_iota(jnp.int32, (TILE, 1), 0))
        rmask = (gidx >= start) & (gidx < end)
        vm = jnp.where(rmask, vt, 0.0)
        vseg_ref[...] += jnp.sum(vm, axis=0, keepdims=True)
        v2_scr[pl.ds(t * TILE, TILE), :] = jnp.zeros((TILE, HEADS * 128),
                                                     jnp.float32)
        rmask_f = rmask.astype(jnp.float32)
        for h in range(HEADS):
            sl = slice(h * HEAD_DIM, (h + 1) * HEAD_DIM)
            v2_scr[pl.ds(t * TILE, TILE), h * 128:h * 128 + 64] = vm[:, sl]
            v2_scr[pl.ds(t * TILE, TILE), h * 128 + 64:h * 128 + 65] = rmask_f
        return 0

    jax.lax.fori_loop(0, nt, kv_body, 0)

    comp_d = (SEQ - length).astype(jnp.float32)

    cp_wq.wait()

    def ti_body(ti, _):
        rows = x_scr[pl.ds(src0 + ti * TILE, TILE), :]
        qt = _dotT(rows, wq_scr[...]) + bq_ref[...]
        num_scr[...] = jnp.zeros_like(num_scr)

        def tj_body(tj, _):
            kt = k_scr[pl.ds(tj * TILE, TILE), :]
            for h in range(HEADS):
                sl = slice(h * HEAD_DIM, (h + 1) * HEAD_DIM)
                s = _dotT(qt[:, sl], kt[:, sl])
                e = jnp.exp(s)
                num_scr[:, h * 128:(h + 1) * 128] += jax.lax.dot_general(
                    e, v2_scr[pl.ds(tj * TILE, TILE),
                              h * 128:(h + 1) * 128],
                    (((1,), (0,)), ((), ())),
                    preferred_element_type=jnp.float32)
            return 0

        jax.lax.fori_loop(0, nt, tj_body, 0)

        @pl.when(ti == 0)
        def _vtot():
            jax.lax.fori_loop(nseg, NUM_TILES,
                              lambda i, c: (xcp(i).wait(), c)[1], 0)
            ones_row = jnp.ones((1, SEQ), jnp.float32)
            sum_x = jax.lax.dot_general(ones_row, x_scr[...],
                                        (((1,), (0,)), ((), ())),
                                        preferred_element_type=jnp.float32)
            vtot = (_dotT(sum_x, wkv_scr[pl.ds(EMBED, EMBED), :])
                    + SEQ * bv_ref[...])
            vseg_ref[...] = vtot - vseg_ref[...]

        comp_v = vseg_ref[...]
        gidx = (src0 + ti * TILE
                + jax.lax.broadcasted_iota(jnp.int32, (TILE, 1), 0))
        rmask = (gidx >= start) & (gidx < end)
        for h in range(HEADS):
            sl = slice(h * HEAD_DIM, (h + 1) * HEAD_DIM)
            w = ((num_scr[:, h * 128:h * 128 + 64] + comp_v[:, sl])
                 / (num_scr[:, h * 128 + 64:h * 128 + 65] + comp_d))
            acc_ref[:, sl] += jnp.sum(jnp.where(rmask, w, 0.0), axis=0,
                                      keepdims=True)
        return 0

    jax.lax.fori_loop(0, nt, ti_body, 0)

    cp_wo.wait()
    mean_w = acc_ref[...] / length.astype(jnp.float32)
    out_ref[...] = _dotT(mean_w, wo_scr[...]) + bo_ref[...]


def kernel(x, segment_ids, pos, Wq, bq, Wk, bk, Wv, bv, Wo, bo):
    seg2d = segment_ids.astype(jnp.int32).reshape(16, 128)
    pos_arr = jnp.asarray(pos, jnp.int32).reshape(1)
    hbm = pl.BlockSpec(memory_space=pltpu.MemorySpace.HBM)
    vmem = pl.BlockSpec(memory_space=pltpu.VMEM)
    out = pl.pallas_call(
        _body,
        out_shape=jax.ShapeDtypeStruct((1, EMBED), jnp.float32),
        in_specs=[
            hbm,
            vmem,
            pl.BlockSpec(memory_space=pltpu.SMEM),
            hbm,
            vmem,
            hbm,
            vmem,
            hbm,
            vmem,
            hbm,
            vmem,
        ],
        out_specs=vmem,
        scratch_shapes=[
            pltpu.VMEM((SEQ, EMBED), jnp.float32),
            pltpu.VMEM((EMBED, EMBED), jnp.float32),
            pltpu.VMEM((2 * EMBED, EMBED), jnp.float32),
            pltpu.VMEM((EMBED, EMBED), jnp.float32),
            pltpu.VMEM((SEQ, EMBED), jnp.float32),
            pltpu.VMEM((SEQ, HEADS * 128), jnp.float32),
            pltpu.VMEM((TILE, HEADS * 128), jnp.float32),
            pltpu.VMEM((1, EMBED), jnp.float32),
            pltpu.VMEM((1, EMBED), jnp.float32),
            pltpu.SemaphoreType.DMA((12,)),
        ],
    )(x, seg2d, pos_arr,
      Wq, bq.reshape(1, EMBED), Wk, bk.reshape(1, EMBED),
      Wv, bv.reshape(1, EMBED), Wo, bo.reshape(1, EMBED))
    return out.reshape(EMBED)


# speedup vs baseline: 1.0589x; 1.0589x over previous
import jax
import jax.numpy as jnp
from jax.experimental import pallas as pl
from jax.experimental.pallas import tpu as pltpu

SEQ = 2048
EMBED = 1024
HEADS = 16
HEAD_DIM = EMBED // HEADS
TILE = 512
NUM_TILES = SEQ // TILE

_DN = (((1,), (1,)), ((), ()))


def _dotT(a, b):
    return jax.lax.dot_general(a, b, _DN, preferred_element_type=jnp.float32)


def _body(x_hbm, seg_ref, pos_ref, wq_hbm, bq_ref, wk_hbm, bk_ref,
          wv_hbm, bv_ref, wo_hbm, bo_ref, out_ref,
          x_scr, wq_scr, wkv_scr, wo_scr,
          k_scr, v2_scr, num_scr, acc_ref, vseg_ref, sems):
    pos = pos_ref[0]
    seg = seg_ref[...]
    flat_idx = (jax.lax.broadcasted_iota(jnp.int32, seg.shape, 0) * 128
                + jax.lax.broadcasted_iota(jnp.int32, seg.shape, 1))
    sid = jnp.sum(jnp.where(flat_idx == pos, seg, 0))
    start = jnp.sum((seg < sid).astype(jnp.int32))
    end = jnp.sum((seg <= sid).astype(jnp.int32))
    length = end - start
    nt = jnp.minimum((length + 7 + TILE - 1) // TILE, NUM_TILES)
    src0 = 8 * jnp.minimum(start // 8, SEQ // 8 - nt * (TILE // 8))
    ta0 = start // TILE
    ta1 = (end - 1) // TILE + 1
    nseg = ta1 - ta0

    def xcp(i):
        t = (ta0 + i) % NUM_TILES
        return pltpu.make_async_copy(
            x_hbm.at[pl.ds(t * TILE, TILE), :],
            x_scr.at[pl.ds(t * TILE, TILE), :], sems.at[i])

    cp_wk = pltpu.make_async_copy(wk_hbm, wkv_scr.at[pl.ds(0, EMBED), :],
                                  sems.at[8])
    cp_wv = pltpu.make_async_copy(wv_hbm, wkv_scr.at[pl.ds(EMBED, EMBED), :],
                                  sems.at[9])
    cp_wq = pltpu.make_async_copy(wq_hbm, wq_scr, sems.at[10])
    cp_wo = pltpu.make_async_copy(wo_hbm, wo_scr, sems.at[11])

    xcp(0).start()
    xcp(1).start()
    cp_wk.start()
    cp_wv.start()
    cp_wq.start()
    for i in range(2, NUM_TILES):
        xcp(i).start()
    cp_wo.start()

    acc_ref[...] = jnp.zeros_like(acc_ref)
    vseg_ref[...] = jnp.zeros_like(vseg_ref)

    jax.lax.fori_loop(0, nseg, lambda i, c: (xcp(i).wait(), c)[1], 0)
    cp_wk.wait()
    cp_wv.wait()

    def kv_body(t, _):
        rows = x_scr[pl.ds(src0 + t * TILE, TILE), :]
        kvt = _dotT(rows, wkv_scr[...])
        kt = kvt[:, 0:EMBED] + bk_ref[...]
        vt = kvt[:, EMBED:2 * EMBED] + bv_ref[...]
        k_scr[pl.ds(t * TILE, TILE), :] = kt
        gidx = (src0 + t * TILE
                + jax.lax.broadcasted_iota(jnp.int32, (TILE, 1), 0))
        rmask = (gidx >= start) & (gidx < end)
        vm = jnp.where(rmask, vt, 0.0)
        vseg_ref[...] += jnp.sum(vm, axis=0, keepdims=True)
        v2_scr[pl.ds(t * TILE, TILE), :] = jnp.zeros((TILE, HEADS * 128),
                                                     jnp.float32)
        rmask_f = rmask.astype(jnp.float32)
        for h in range(HEADS):
            sl = slice(h * HEAD_DIM, (h + 1) * HEAD_DIM)
            v2_scr[pl.ds(t * TILE, TILE), h * 128:h * 128 + 64] = vm[:, sl]
            v2_scr[pl.ds(t * TILE, TILE), h * 128 + 64:h * 128 + 65] = rmask_f
        return 0

    jax.lax.fori_loop(0, nt, kv_body, 0)

    comp_d = (SEQ - length).astype(jnp.float32)

    cp_wq.wait()

    def ti_body(ti, _):
        rows = x_scr[pl.ds(src0 + ti * TILE, TILE), :]
        qt = _dotT(rows, wq_scr[...]) + bq_ref[...]
        num_scr[...] = jnp.zeros_like(num_scr)

        def tj_body(tj, _):
            kt = k_scr[pl.ds(tj * TILE, TILE), :]
            for h in range(HEADS):
                sl = slice(h * HEAD_DIM, (h + 1) * HEAD_DIM)
                s = _dotT(qt[:, sl], kt[:, sl])
                e = jnp.exp(s)
                num_scr[:, h * 128:(h + 1) * 128] += jax.lax.dot_general(
                    e, v2_scr[pl.ds(tj * TILE, TILE),
                              h * 128:(h + 1) * 128],
                    (((1,), (0,)), ((), ())),
                    preferred_element_type=jnp.float32)
            return 0

        jax.lax.fori_loop(0, nt, tj_body, 0)

        @pl.when(ti == 0)
        def _vtot():
            jax.lax.fori_loop(nseg, NUM_TILES,
                              lambda i, c: (xcp(i).wait(), c)[1], 0)
            ones_row = jnp.ones((1, SEQ), jnp.float32)
            sum_x = jax.lax.dot_general(ones_row, x_scr[...],
                                        (((1,), (0,)), ((), ())),
                                        preferred_element_type=jnp.float32)
            vtot = (_dotT(sum_x, wkv_scr[pl.ds(EMBED, EMBED), :])
                    + SEQ * bv_ref[...])
            vseg_ref[...] = vtot - vseg_ref[...]

        comp_v = vseg_ref[...]
        gidx = (src0 + ti * TILE
                + jax.lax.broadcasted_iota(jnp.int32, (TILE, 1), 0))
        rmask = (gidx >= start) & (gidx < end)
        for h in range(HEADS):
            sl = slice(h * HEAD_DIM, (h + 1) * HEAD_DIM)
            w = ((num_scr[:, h * 128:h * 128 + 64] + comp_v[:, sl])
                 / (num_scr[:, h * 128 + 64:h * 128 + 65] + comp_d))
            acc_ref[:, sl] += jnp.sum(jnp.where(rmask, w, 0.0), axis=0,
                                      keepdims=True)
        return 0

    jax.lax.fori_loop(0, nt, ti_body, 0)

    cp_wo.wait()
    mean_w = acc_ref[...] / length.astype(jnp.float32)
    out_ref[...] = _dotT(mean_w, wo_scr[...]) + bo_ref[...]


def kernel(x, segment_ids, pos, Wq, bq, Wk, bk, Wv, bv, Wo, bo):
    seg2d = segment_ids.astype(jnp.int32).reshape(16, 128)
    pos_arr = jnp.asarray(pos, jnp.int32).reshape(1)
    hbm = pl.BlockSpec(memory_space=pltpu.MemorySpace.HBM)
    vmem = pl.BlockSpec(memory_space=pltpu.VMEM)
    out = pl.pallas_call(
        _body,
        out_shape=jax.ShapeDtypeStruct((1, EMBED), jnp.float32),
        in_specs=[
            hbm,
            vmem,
            pl.BlockSpec(memory_space=pltpu.SMEM),
            hbm,
            vmem,
            hbm,
            vmem,
            hbm,
            vmem,
            hbm,
            vmem,
        ],
        out_specs=vmem,
        scratch_shapes=[
            pltpu.VMEM((SEQ, EMBED), jnp.float32),
            pltpu.VMEM((EMBED, EMBED), jnp.float32),
            pltpu.VMEM((2 * EMBED, EMBED), jnp.float32),
            pltpu.VMEM((EMBED, EMBED), jnp.float32),
            pltpu.VMEM((SEQ, EMBED), jnp.float32),
            pltpu.VMEM((SEQ, HEADS * 128), jnp.float32),
            pltpu.VMEM((TILE, HEADS * 128), jnp.float32),
            pltpu.VMEM((1, EMBED), jnp.float32),
            pltpu.VMEM((1, EMBED), jnp.float32),
            pltpu.SemaphoreType.DMA((12,)),
        ],
    )(x, seg2d, pos_arr,
      Wq, bq.reshape(1, EMBED), Wk, bk.reshape(1, EMBED),
      Wv, bv.reshape(1, EMBED), Wo, bo.reshape(1, EMBED))
    return out.reshape(EMBED)


# speedup vs baseline: 1.1558x; 1.0915x over previous
import jax
import jax.numpy as jnp
from jax.experimental import pallas as pl
from jax.experimental.pallas import tpu as pltpu

SEQ = 2048
EMBED = 1024
HEADS = 16
HEAD_DIM = EMBED // HEADS
TILE = 512
NUM_TILES = SEQ // TILE

_DN = (((1,), (1,)), ((), ()))


def _dotT(a, b):
    return jax.lax.dot_general(a, b, _DN, preferred_element_type=jnp.float32)


def _body(x_hbm, seg_ref, pos_ref, wq_hbm, bq_ref, wk_hbm, bk_ref,
          wv_hbm, bv_ref, wo_hbm, bo_ref, out_ref,
          x_scr, wq_scr, wkv_scr, wo_scr,
          k_scr, v2_scr, num_scr, acc_ref, vseg_ref, sems):
    pos = pos_ref[0]
    seg = seg_ref[...]
    flat_idx = (jax.lax.broadcasted_iota(jnp.int32, seg.shape, 0) * 128
                + jax.lax.broadcasted_iota(jnp.int32, seg.shape, 1))
    sid = jnp.sum(jnp.where(flat_idx == pos, seg, 0))
    start = jnp.sum((seg < sid).astype(jnp.int32))
    end = jnp.sum((seg <= sid).astype(jnp.int32))
    length = end - start
    nt = jnp.minimum((length + 7 + TILE - 1) // TILE, NUM_TILES)
    src0 = 8 * jnp.minimum(start // 8, SEQ // 8 - nt * (TILE // 8))

    def xscp(t):
        return pltpu.make_async_copy(
            x_hbm.at[pl.ds(src0 + t * TILE, TILE), :],
            x_scr.at[pl.ds(src0 + t * TILE, TILE), :], sems.at[t])

    def wkvcp(c):
        src = wk_hbm if c < 2 else wv_hbm
        return pltpu.make_async_copy(
            src.at[pl.ds((c % 2) * 512, 512), :],
            wkv_scr.at[pl.ds(c * 512, 512), :], sems.at[4 + c])

    cp_xfull = pltpu.make_async_copy(x_hbm, x_scr, sems.at[9])
    cp_wq = pltpu.make_async_copy(wq_hbm, wq_scr, sems.at[10])
    cp_wo = pltpu.make_async_copy(wo_hbm, wo_scr, sems.at[11])

    xscp(0).start()
    for c in range(4):
        wkvcp(c).start()
    cp_wq.start()
    for t in range(1, NUM_TILES):
        @pl.when(t < nt)
        def _(t=t):
            xscp(t).start()
    cp_xfull.start()
    cp_wo.start()

    acc_ref[...] = jnp.zeros_like(acc_ref)
    vseg_ref[...] = jnp.zeros_like(vseg_ref)

    xscp(0).wait()

    def finish_kv(t, kt, vt):
        k_scr[pl.ds(t * TILE, TILE), :] = kt
        gidx = (src0 + t * TILE
                + jax.lax.broadcasted_iota(jnp.int32, (TILE, 1), 0))
        rmask = (gidx >= start) & (gidx < end)
        vm = jnp.where(rmask, vt, 0.0)
        vseg_ref[...] += jnp.sum(vm, axis=0, keepdims=True)
        v2_scr[pl.ds(t * TILE, TILE), :] = jnp.zeros((TILE, HEADS * 128),
                                                     jnp.float32)
        rmask_f = rmask.astype(jnp.float32)
        for h in range(HEADS):
            sl = slice(h * HEAD_DIM, (h + 1) * HEAD_DIM)
            v2_scr[pl.ds(t * TILE, TILE), h * 128:h * 128 + 64] = vm[:, sl]
            v2_scr[pl.ds(t * TILE, TILE), h * 128 + 64:h * 128 + 65] = rmask_f

    rows0 = x_scr[pl.ds(src0, TILE), :]
    gidx0 = src0 + jax.lax.broadcasted_iota(jnp.int32, (TILE, 1), 0)
    rmask0 = (gidx0 >= start) & (gidx0 < end)
    rmask0_f = rmask0.astype(jnp.float32)
    v2_scr[pl.ds(0, TILE), :] = jnp.zeros((TILE, HEADS * 128), jnp.float32)
    for c in range(4):
        wkvcp(c).wait()
        half = slice((c % 2) * 512, (c % 2) * 512 + 512)
        part = _dotT(rows0, wkv_scr[pl.ds(c * 512, 512), :])
        if c < 2:
            k_scr[pl.ds(0, TILE), half] = part + bk_ref[:, half]
        else:
            vm = jnp.where(rmask0, part + bv_ref[:, half], 0.0)
            vseg_ref[:, half] += jnp.sum(vm, axis=0, keepdims=True)
            for hh in range(8):
                h = (c - 2) * 8 + hh
                v2_scr[pl.ds(0, TILE), h * 128:h * 128 + 64] = (
                    vm[:, hh * 64:hh * 64 + 64])
                v2_scr[pl.ds(0, TILE), h * 128 + 64:h * 128 + 65] = rmask0_f

    def kv_body(t, _):
        xscp(t).wait()
        rows = x_scr[pl.ds(src0 + t * TILE, TILE), :]
        kvt = _dotT(rows, wkv_scr[...])
        finish_kv(t, kvt[:, 0:EMBED] + bk_ref[...],
                  kvt[:, EMBED:2 * EMBED] + bv_ref[...])
        return 0

    jax.lax.fori_loop(1, nt, kv_body, 0)

    comp_d = (SEQ - length).astype(jnp.float32)

    cp_wq.wait()

    def ti_body(ti, _):
        rows = x_scr[pl.ds(src0 + ti * TILE, TILE), :]
        qt = _dotT(rows, wq_scr[...]) + bq_ref[...]
        num_scr[...] = jnp.zeros_like(num_scr)

        def tj_body(tj, _):
            kt = k_scr[pl.ds(tj * TILE, TILE), :]
            for h in range(HEADS):
                sl = slice(h * HEAD_DIM, (h + 1) * HEAD_DIM)
                s = _dotT(qt[:, sl], kt[:, sl])
                e = jnp.exp(s)
                num_scr[:, h * 128:(h + 1) * 128] += jax.lax.dot_general(
                    e, v2_scr[pl.ds(tj * TILE, TILE),
                              h * 128:(h + 1) * 128],
                    (((1,), (0,)), ((), ())),
                    preferred_element_type=jnp.float32)
            return 0

        jax.lax.fori_loop(0, nt, tj_body, 0)

        @pl.when(ti == 0)
        def _vtot():
            cp_xfull.wait()
            ones_row = jnp.ones((1, SEQ), jnp.float32)
            sum_x = jax.lax.dot_general(ones_row, x_scr[...],
                                        (((1,), (0,)), ((), ())),
                                        preferred_element_type=jnp.float32)
            vtot = (_dotT(sum_x, wkv_scr[pl.ds(EMBED, EMBED), :])
                    + SEQ * bv_ref[...])
            vseg_ref[...] = vtot - vseg_ref[...]

        comp_v = vseg_ref[...]
        gidx = (src0 + ti * TILE
                + jax.lax.broadcasted_iota(jnp.int32, (TILE, 1), 0))
        rmask = (gidx >= start) & (gidx < end)
        for h in range(HEADS):
            sl = slice(h * HEAD_DIM, (h + 1) * HEAD_DIM)
            w = ((num_scr[:, h * 128:h * 128 + 64] + comp_v[:, sl])
                 / (num_scr[:, h * 128 + 64:h * 128 + 65] + comp_d))
            acc_ref[:, sl] += jnp.sum(jnp.where(rmask, w, 0.0), axis=0,
                                      keepdims=True)
        return 0

    jax.lax.fori_loop(0, nt, ti_body, 0)

    cp_wo.wait()
    mean_w = acc_ref[...] / length.astype(jnp.float32)
    out_ref[...] = _dotT(mean_w, wo_scr[...]) + bo_ref[...]


def kernel(x, segment_ids, pos, Wq, bq, Wk, bk, Wv, bv, Wo, bo):
    seg2d = segment_ids.astype(jnp.int32).reshape(16, 128)
    pos_arr = jnp.asarray(pos, jnp.int32).reshape(1)
    hbm = pl.BlockSpec(memory_space=pltpu.MemorySpace.HBM)
    vmem = pl.BlockSpec(memory_space=pltpu.VMEM)
    out = pl.pallas_call(
        _body,
        out_shape=jax.ShapeDtypeStruct((1, EMBED), jnp.float32),
        in_specs=[
            hbm,
            vmem,
            pl.BlockSpec(memory_space=pltpu.SMEM),
            hbm,
            vmem,
            hbm,
            vmem,
            hbm,
            vmem,
            hbm,
            vmem,
        ],
        out_specs=vmem,
        scratch_shapes=[
            pltpu.VMEM((SEQ, EMBED), jnp.float32),
            pltpu.VMEM((EMBED, EMBED), jnp.float32),
            pltpu.VMEM((2 * EMBED, EMBED), jnp.float32),
            pltpu.VMEM((EMBED, EMBED), jnp.float32),
            pltpu.VMEM((SEQ, EMBED), jnp.float32),
            pltpu.VMEM((SEQ, HEADS * 128), jnp.float32),
            pltpu.VMEM((TILE, HEADS * 128), jnp.float32),
            pltpu.VMEM((1, EMBED), jnp.float32),
            pltpu.VMEM((1, EMBED), jnp.float32),
            pltpu.SemaphoreType.DMA((12,)),
        ],
    )(x, seg2d, pos_arr,
      Wq, bq.reshape(1, EMBED), Wk, bk.reshape(1, EMBED),
      Wv, bv.reshape(1, EMBED), Wo, bo.reshape(1, EMBED))
    return out.reshape(EMBED)
